# baseline (speedup 1.0000x reference)
import jax
import jax.numpy as jnp
import numpy as np
from jax.experimental import pallas as pl
from jax.experimental.pallas import tpu as pltpu

WP = 6
WF = 6
ROWB = 128
KWIN = 256
BSTEP = 2


def _edge_att_kernel(lens_ref, nf_ref, w_ref, out_ref):
    bs = pl.program_id(0)
    L = nf_ref.shape[1]
    nt = (((1,), (1,)), ((), ()))
    scale = np.float32(1.0 / np.sqrt(200.0))
    ij = jax.lax.broadcasted_iota(jnp.int32, (ROWB, KWIN), 0)
    ik = jax.lax.broadcasted_iota(jnp.int32, (ROWB, KWIN), 1)
    diff = ik - ij
    zeros_row = jnp.zeros((ROWB, L), jnp.float32)
    for u in range(BSTEP):
        nf = nf_ref[u]
        att = jax.lax.dot_general(nf, w_ref[...], nt,
                                  preferred_element_type=jnp.float32) * scale
        n = lens_ref[bs * BSTEP + u]
        for r in range(L // ROWB):
            j0 = ROWB * r
            start = min(max(j0 - 64, 0), L - KWIN)
            off = j0 - start
            scores = jax.lax.dot_general(
                nf[j0:j0 + ROWB], att[start:start + KWIN], nt,
                preferred_element_type=jnp.float32)
            mask = ((diff >= off - WP) & (diff <= off + WF)
                    & (ik < n - start) & (ij < n - j0))
            e = jnp.where(mask, jnp.exp(scores), jnp.float32(0.0))
            s = jnp.sum(e, axis=1, keepdims=True)
            p = e * jnp.where(s > 0, 1.0 / s, jnp.float32(0.0))
            out_ref[u, j0:j0 + ROWB, :] = zeros_row
            out_ref[u, j0:j0 + ROWB, start:start + KWIN] = p


def kernel(node_features, node_num_tensor, weight):
    B, L, G = node_features.shape
    lens = node_num_tensor.astype(jnp.int32)
    grid_spec = pltpu.PrefetchScalarGridSpec(
        num_scalar_prefetch=1,
        grid=(B // BSTEP,),
        in_specs=[
            pl.BlockSpec((BSTEP, L, G), lambda b, lens_ref: (b, 0, 0)),
            pl.BlockSpec((G, G), lambda b, lens_ref: (0, 0)),
        ],
        out_specs=pl.BlockSpec((BSTEP, L, L), lambda b, lens_ref: (b, 0, 0)),
    )
    return pl.pallas_call(
        _edge_att_kernel,
        grid_spec=grid_spec,
        out_shape=jax.ShapeDtypeStruct((B, L, L), jnp.float32),
        compiler_params=pltpu.CompilerParams(
            dimension_semantics=("arbitrary",),
        ),
    )(lens, node_features, weight)


# speedup vs baseline: 1.1694x; 1.1694x over previous
import jax
import jax.numpy as jnp
import numpy as np
from jax.experimental import pallas as pl
from jax.experimental.pallas import tpu as pltpu

WP = 6
WF = 6
ROWB = 128
KWIN = 384
BSTEP = 4


def _edge_att_kernel(lens_ref, nf_ref, w_ref, out_ref):
    bs = pl.program_id(0)
    L = nf_ref.shape[1]
    nt = (((1,), (1,)), ((), ()))
    scale = np.float32(1.0 / np.sqrt(200.0))
    ij = jax.lax.broadcasted_iota(jnp.int32, (ROWB, KWIN), 0)
    ik = jax.lax.broadcasted_iota(jnp.int32, (ROWB, KWIN), 1)
    diff = ik - ij
    zeros_comp = jnp.zeros((ROWB, L - KWIN), jnp.float32)
    for u in range(BSTEP):
        nf = nf_ref[u]
        att = jax.lax.dot_general(nf, w_ref[...], nt,
                                  preferred_element_type=jnp.float32) * scale
        n = lens_ref[bs * BSTEP + u]
        for r in range(L // ROWB):
            j0 = ROWB * r
            start = min(max(ROWB * (r - 1), 0), L - KWIN)
            off = j0 - start
            scores = jax.lax.dot_general(
                nf[j0:j0 + ROWB], att[start:start + KWIN], nt,
                preferred_element_type=jnp.float32)
            mask = ((diff >= off - WP) & (diff <= off + WF)
                    & (ik < n - start) & (ij < n - j0))
            e = jnp.where(mask, jnp.exp(scores), jnp.float32(0.0))
            s = jnp.sum(e, axis=1, keepdims=True)
            p = e * jnp.where(s > 0, 1.0 / s, jnp.float32(0.0))
            out_ref[u, j0:j0 + ROWB, start:start + KWIN] = p
            comp = KWIN if start == 0 else 0
            out_ref[u, j0:j0 + ROWB, comp:comp + (L - KWIN)] = zeros_comp


def kernel(node_features, node_num_tensor, weight):
    B, L, G = node_features.shape
    lens = node_num_tensor.astype(jnp.int32)
    grid_spec = pltpu.PrefetchScalarGridSpec(
        num_scalar_prefetch=1,
        grid=(B // BSTEP,),
        in_specs=[
            pl.BlockSpec((BSTEP, L, G), lambda b, lens_ref: (b, 0, 0)),
            pl.BlockSpec((G, G), lambda b, lens_ref: (0, 0)),
        ],
        out_specs=pl.BlockSpec((BSTEP, L, L), lambda b, lens_ref: (b, 0, 0)),
    )
    return pl.pallas_call(
        _edge_att_kernel,
        grid_spec=grid_spec,
        out_shape=jax.ShapeDtypeStruct((B, L, L), jnp.float32),
        compiler_params=pltpu.CompilerParams(
            dimension_semantics=("arbitrary",),
        ),
    )(lens, node_features, weight)


# speedup vs baseline: 1.1719x; 1.0021x over previous
import jax
import jax.numpy as jnp
import numpy as np
from jax.experimental import pallas as pl
from jax.experimental.pallas import tpu as pltpu

WP = 6
WF = 6
ROWB = 128
KWIN = 384
BSTEP = 4


def _edge_att_kernel(lens_ref, nf_ref, w_ref, out_ref):
    bs = pl.program_id(0)
    L = nf_ref.shape[1]
    nt = (((1,), (1,)), ((), ()))
    scale = np.float32(1.0 / np.sqrt(200.0))
    ij = jax.lax.broadcasted_iota(jnp.int32, (ROWB, KWIN), 0)
    ik = jax.lax.broadcasted_iota(jnp.int32, (ROWB, KWIN), 1)
    diff = ik - ij
    zeros_comp = jnp.zeros((ROWB, L - KWIN), jnp.float32)
    for u in range(BSTEP):
        nf = nf_ref[u]
        nfh = nf.astype(jnp.bfloat16)
        att = jax.lax.dot_general(nfh, w_ref[...].astype(jnp.bfloat16), nt,
                                  preferred_element_type=jnp.float32) * scale
        atth = att.astype(jnp.bfloat16)
        n = lens_ref[bs * BSTEP + u]
        for r in range(L // ROWB):
            j0 = ROWB * r
            start = min(max(ROWB * (r - 1), 0), L - KWIN)
            off = j0 - start
            scores = jax.lax.dot_general(
                nfh[j0:j0 + ROWB], atth[start:start + KWIN], nt,
                preferred_element_type=jnp.float32)
            mask = ((diff >= off - WP) & (diff <= off + WF)
                    & (ik < n - start) & (ij < n - j0))
            e = jnp.where(mask, jnp.exp(scores), jnp.float32(0.0))
            s = jnp.sum(e, axis=1, keepdims=True)
            p = e * jnp.where(s > 0, 1.0 / s, jnp.float32(0.0))
            out_ref[u, j0:j0 + ROWB, start:start + KWIN] = p
            comp = KWIN if start == 0 else 0
            out_ref[u, j0:j0 + ROWB, comp:comp + (L - KWIN)] = zeros_comp


def kernel(node_features, node_num_tensor, weight):
    B, L, G = node_features.shape
    lens = node_num_tensor.astype(jnp.int32)
    grid_spec = pltpu.PrefetchScalarGridSpec(
        num_scalar_prefetch=1,
        grid=(B // BSTEP,),
        in_specs=[
            pl.BlockSpec((BSTEP, L, G), lambda b, lens_ref: (b, 0, 0)),
            pl.BlockSpec((G, G), lambda b, lens_ref: (0, 0)),
        ],
        out_specs=pl.BlockSpec((BSTEP, L, L), lambda b, lens_ref: (b, 0, 0)),
    )
    return pl.pallas_call(
        _edge_att_kernel,
        grid_spec=grid_spec,
        out_shape=jax.ShapeDtypeStruct((B, L, L), jnp.float32),
        compiler_params=pltpu.CompilerParams(
            dimension_semantics=("arbitrary",),
        ),
    )(lens, node_features, weight)


# speedup vs baseline: 1.3261x; 1.1316x over previous
import jax
import jax.numpy as jnp
import numpy as np
from jax.experimental import pallas as pl
from jax.experimental.pallas import tpu as pltpu

WP = 6
WF = 6
ROWB = 256
KWIN = 384
BSTEP = 4


def _edge_att_kernel(lens_ref, nf_ref, w_ref, out_ref):
    bs = pl.program_id(0)
    L = nf_ref.shape[1]
    nt = (((1,), (1,)), ((), ()))
    scale = np.float32(1.0 / np.sqrt(200.0))
    ij = jax.lax.broadcasted_iota(jnp.int32, (ROWB, KWIN), 0)
    ik = jax.lax.broadcasted_iota(jnp.int32, (ROWB, KWIN), 1)
    diff = ik - ij
    zeros_comp = jnp.zeros((ROWB, L - KWIN), jnp.float32)
    for u in range(BSTEP):
        nf = nf_ref[u]
        att = jax.lax.dot_general(nf, w_ref[...], nt,
                                  preferred_element_type=jnp.float32) * scale
        n = lens_ref[bs * BSTEP + u]
        for r in range(L // ROWB):
            j0 = ROWB * r
            start = min(max(ROWB * (r - 1), 0), L - KWIN)
            off = j0 - start
            scores = jax.lax.dot_general(
                nf[j0:j0 + ROWB], att[start:start + KWIN], nt,
                preferred_element_type=jnp.float32)
            mask = ((diff >= off - WP) & (diff <= off + WF)
                    & (ik < n - start) & (ij < n - j0))
            e = jnp.where(mask, jnp.exp(scores), jnp.float32(0.0))
            s = jnp.sum(e, axis=1, keepdims=True)
            p = e * jnp.where(s > 0, 1.0 / s, jnp.float32(0.0))
            out_ref[u, j0:j0 + ROWB, start:start + KWIN] = p
            comp = KWIN if start == 0 else 0
            out_ref[u, j0:j0 + ROWB, comp:comp + (L - KWIN)] = zeros_comp


def kernel(node_features, node_num_tensor, weight):
    B, L, G = node_features.shape
    lens = node_num_tensor.astype(jnp.int32)
    grid_spec = pltpu.PrefetchScalarGridSpec(
        num_scalar_prefetch=1,
        grid=(B // BSTEP,),
        in_specs=[
            pl.BlockSpec((BSTEP, L, G), lambda b, lens_ref: (b, 0, 0)),
            pl.BlockSpec((G, G), lambda b, lens_ref: (0, 0)),
        ],
        out_specs=pl.BlockSpec((BSTEP, L, L), lambda b, lens_ref: (b, 0, 0)),
    )
    return pl.pallas_call(
        _edge_att_kernel,
        grid_spec=grid_spec,
        out_shape=jax.ShapeDtypeStruct((B, L, L), jnp.float32),
        compiler_params=pltpu.CompilerParams(
            dimension_semantics=("arbitrary",),
        ),
    )(lens, node_features, weight)


# speedup vs baseline: 1.3379x; 1.0089x over previous
import jax
import jax.numpy as jnp
import numpy as np
from jax.experimental import pallas as pl
from jax.experimental.pallas import tpu as pltpu

WP = 6
WF = 6
ROWB = 256
KWIN = 384
BSTEP = 4


def _edge_att_kernel(lens_ref, nf_ref, w_ref, out_ref):
    bs = pl.program_id(0)
    L = nf_ref.shape[1]
    nt = (((1,), (1,)), ((), ()))
    scale = np.float32(1.0 / np.sqrt(200.0))
    ij = jax.lax.broadcasted_iota(jnp.int32, (ROWB, KWIN), 0)
    ik = jax.lax.broadcasted_iota(jnp.int32, (ROWB, KWIN), 1)
    diff = ik - ij
    zeros_comp = jnp.zeros((ROWB, L - KWIN), jnp.float32)
    for u in range(BSTEP):
        nf = nf_ref[u]
        att = jax.lax.dot_general(nf, w_ref[...], nt,
                                  preferred_element_type=jnp.float32) * scale
        n = lens_ref[bs * BSTEP + u]
        for r in range(L // ROWB):
            j0 = ROWB * r
            start = min(max(j0 - 128, 0), L - KWIN)
            off = j0 - start
            scores = jax.lax.dot_general(
                nf[j0:j0 + ROWB], att[start:start + KWIN], nt,
                preferred_element_type=jnp.float32)
            mask = ((diff >= off - WP) & (diff <= off + WF)
                    & (ik < n - start) & (ij < n - j0))
            e = jnp.where(mask, jnp.exp(scores), jnp.float32(0.0))
            s = jnp.sum(e, axis=1, keepdims=True)
            p = e * jnp.where(s > 0, 1.0 / s, jnp.float32(0.0))
            out_ref[u, j0:j0 + ROWB, start:start + KWIN] = p
            comp = KWIN if start == 0 else 0
            out_ref[u, j0:j0 + ROWB, comp:comp + (L - KWIN)] = zeros_comp


def kernel(node_features, node_num_tensor, weight):
    B, L, G = node_features.shape
    lens = node_num_tensor.astype(jnp.int32)
    grid_spec = pltpu.PrefetchScalarGridSpec(
        num_scalar_prefetch=1,
        grid=(B // BSTEP,),
        in_specs=[
            pl.BlockSpec((BSTEP, L, G), lambda b, lens_ref: (b, 0, 0)),
            pl.BlockSpec((G, G), lambda b, lens_ref: (0, 0)),
        ],
        out_specs=pl.BlockSpec((BSTEP, L, L), lambda b, lens_ref: (b, 0, 0)),
    )
    return pl.pallas_call(
        _edge_att_kernel,
        grid_spec=grid_spec,
        out_shape=jax.ShapeDtypeStruct((B, L, L), jnp.float32),
        compiler_params=pltpu.CompilerParams(
            dimension_semantics=("arbitrary",),
        ),
    )(lens, node_features, weight)


# speedup vs baseline: 1.3751x; 1.0278x over previous
import jax
import jax.numpy as jnp
import numpy as np
from jax.experimental import pallas as pl
from jax.experimental.pallas import tpu as pltpu

WP = 6
WF = 6
ROWB = 256
KWIN = 384
BSTEP = 8


def _edge_att_kernel(lens_ref, nf_ref, w_ref, out_ref):
    bs = pl.program_id(0)
    L = nf_ref.shape[1]
    nt = (((1,), (1,)), ((), ()))
    scale = np.float32(1.0 / np.sqrt(200.0))
    ij = jax.lax.broadcasted_iota(jnp.int32, (ROWB, KWIN), 0)
    ik = jax.lax.broadcasted_iota(jnp.int32, (ROWB, KWIN), 1)
    diff = ik - ij
    zeros_comp = jnp.zeros((ROWB, L - KWIN), jnp.float32)
    for u in range(BSTEP):
        nf = nf_ref[u]
        att = jax.lax.dot_general(nf, w_ref[...], nt,
                                  preferred_element_type=jnp.float32) * scale
        n = lens_ref[bs * BSTEP + u]
        for r in range(L // ROWB):
            j0 = ROWB * r
            start = min(max(j0 - 128, 0), L - KWIN)
            off = j0 - start
            scores = jax.lax.dot_general(
                nf[j0:j0 + ROWB], att[start:start + KWIN], nt,
                preferred_element_type=jnp.float32)
            mask = ((diff >= off - WP) & (diff <= off + WF)
                    & (ik < n - start) & (ij < n - j0))
            e = jnp.where(mask, jnp.exp(scores), jnp.float32(0.0))
            s = jnp.sum(e, axis=1, keepdims=True)
            p = e * jnp.where(s > 0, 1.0 / s, jnp.float32(0.0))
            out_ref[u, j0:j0 + ROWB, start:start + KWIN] = p
            comp = KWIN if start == 0 else 0
            out_ref[u, j0:j0 + ROWB, comp:comp + (L - KWIN)] = zeros_comp


def kernel(node_features, node_num_tensor, weight):
    B, L, G = node_features.shape
    lens = node_num_tensor.astype(jnp.int32)
    grid_spec = pltpu.PrefetchScalarGridSpec(
        num_scalar_prefetch=1,
        grid=(B // BSTEP,),
        in_specs=[
            pl.BlockSpec((BSTEP, L, G), lambda b, lens_ref: (b, 0, 0)),
            pl.BlockSpec((G, G), lambda b, lens_ref: (0, 0)),
        ],
        out_specs=pl.BlockSpec((BSTEP, L, L), lambda b, lens_ref: (b, 0, 0)),
    )
    return pl.pallas_call(
        _edge_att_kernel,
        grid_spec=grid_spec,
        out_shape=jax.ShapeDtypeStruct((B, L, L), jnp.float32),
        compiler_params=pltpu.CompilerParams(
            dimension_semantics=("arbitrary",),
        ),
    )(lens, node_features, weight)
